# baseline (speedup 1.0000x reference)
import jax
import jax.numpy as jnp
from jax.experimental import pallas as pl

B = 512
N = 64
D = 512
O = 128
BB = 64


def _fused_kernel(states_ref, adj_ref, wg_ref, bg_ref, wcs_ref, wca_ref,
                  bc_ref, outs_ref, vals_ref):
    states = states_ref[...]
    adj = adj_ref[...]

    msg = jax.lax.dot_general(
        states.reshape(BB * N, D), wg_ref[...],
        (((1,), (0,)), ((), ())),
        preferred_element_type=jnp.float32).reshape(BB, N, O)

    eye = (jax.lax.broadcasted_iota(jnp.int32, (N, N), 0) ==
           jax.lax.broadcasted_iota(jnp.int32, (N, N), 1)).astype(jnp.float32)
    a_hat = adj + eye[None, :, :]
    deg = jnp.sum(a_hat, axis=1)
    dinv = jnp.where(deg > 0, jax.lax.rsqrt(deg), 0.0)
    aw = a_hat * dinv[:, :, None] * dinv[:, None, :]

    out = jax.lax.dot_general(
        aw, msg, (((1,), (1,)), ((0,), (0,))),
        preferred_element_type=jnp.float32)
    outs_ref[...] = out + bg_ref[...][None, None, :]

    v_s = jnp.sum(states * wcs_ref[...][None, :, :], axis=(1, 2))
    v_a = jnp.sum(adj * wca_ref[...][None, :, :], axis=(1, 2))
    vals_ref[...] = (v_s + v_a + bc_ref[0, 0])[:, None]


def kernel(obs, W_gnn, b_gnn, W_crit, b_crit):
    states = obs[:, : N * D].reshape(B, N, D)
    adj = obs[:, N * D:].reshape(B, N, N)
    wc = W_crit.reshape(-1)
    wcs = wc[: N * D].reshape(N, D)
    wca = wc[N * D:].reshape(N, N)
    bc = b_crit.reshape(1, 1)

    grid = (B // BB,)
    out3, values = pl.pallas_call(
        _fused_kernel,
        grid=grid,
        in_specs=[
            pl.BlockSpec((BB, N, D), lambda i: (i, 0, 0)),
            pl.BlockSpec((BB, N, N), lambda i: (i, 0, 0)),
            pl.BlockSpec((D, O), lambda i: (0, 0)),
            pl.BlockSpec((O,), lambda i: (0,)),
            pl.BlockSpec((N, D), lambda i: (0, 0)),
            pl.BlockSpec((N, N), lambda i: (0, 0)),
            pl.BlockSpec((1, 1), lambda i: (0, 0)),
        ],
        out_specs=[
            pl.BlockSpec((BB, N, O), lambda i: (i, 0, 0)),
            pl.BlockSpec((BB, 1), lambda i: (i, 0)),
        ],
        out_shape=[
            jax.ShapeDtypeStruct((B, N, O), jnp.float32),
            jax.ShapeDtypeStruct((B, 1), jnp.float32),
        ],
    )(states, adj, W_gnn, b_gnn, wcs, wca, bc)
    return out3.reshape(B, N * O), values


# speedup vs baseline: 3.2665x; 3.2665x over previous
import jax
import jax.numpy as jnp
from jax.experimental import pallas as pl

B = 512
N = 64
D = 512
O = 128
BB = 64


def _fused_kernel(states_ref, adj_ref, wg_ref, bg_ref, wcs_ref, wca_ref,
                  bc_ref, outs_ref, vals_ref):
    st = states_ref[...]
    adjf = adj_ref[...]

    msg = jax.lax.dot_general(
        st.reshape(BB * N, D), wg_ref[...],
        (((1,), (0,)), ((), ())),
        preferred_element_type=jnp.float32).reshape(BB, N, O)

    adj = adjf.reshape(BB, N, N)
    eye = (jax.lax.broadcasted_iota(jnp.int32, (N, N), 0) ==
           jax.lax.broadcasted_iota(jnp.int32, (N, N), 1)).astype(jnp.float32)
    a_hat = adj + eye[None, :, :]
    deg = jnp.sum(a_hat, axis=1)
    dinv = jnp.where(deg > 0, jax.lax.rsqrt(deg), 0.0)
    aw = a_hat * dinv[:, :, None] * dinv[:, None, :]

    out = jax.lax.dot_general(
        aw, msg, (((1,), (1,)), ((0,), (0,))),
        preferred_element_type=jnp.float32)
    outs_ref[...] = (out + bg_ref[...][None, None, :]).reshape(BB, N * O)

    v_s = jnp.sum(st * wcs_ref[...], axis=1)
    v_a = jnp.sum(adjf * wca_ref[...], axis=1)
    vals_ref[...] = (v_s + v_a + bc_ref[0, 0])[:, None]


def kernel(obs, W_gnn, b_gnn, W_crit, b_crit):
    wc = W_crit.reshape(1, -1)
    wcs = wc[:, : N * D]
    wca = wc[:, N * D:]
    bc = b_crit.reshape(1, 1)

    grid = (B // BB,)
    outs, values = pl.pallas_call(
        _fused_kernel,
        grid=grid,
        in_specs=[
            pl.BlockSpec((BB, N * D), lambda i: (i, 0)),
            pl.BlockSpec((BB, N * N), lambda i: (i, N * D // (N * N))),
            pl.BlockSpec((D, O), lambda i: (0, 0)),
            pl.BlockSpec((O,), lambda i: (0,)),
            pl.BlockSpec((1, N * D), lambda i: (0, 0)),
            pl.BlockSpec((1, N * N), lambda i: (0, 0)),
            pl.BlockSpec((1, 1), lambda i: (0, 0)),
        ],
        out_specs=[
            pl.BlockSpec((BB, N * O), lambda i: (i, 0)),
            pl.BlockSpec((BB, 1), lambda i: (i, 0)),
        ],
        out_shape=[
            jax.ShapeDtypeStruct((B, N * O), jnp.float32),
            jax.ShapeDtypeStruct((B, 1), jnp.float32),
        ],
    )(obs, obs, W_gnn, b_gnn, wcs, wca, bc)
    return outs, values
